# Initial kernel scaffold; baseline (speedup 1.0000x reference)
import jax
import jax.numpy as jnp
from jax.experimental import pallas as pl


def kernel(hidden_states, gate_weight, e_score_correction_bias, W1, W3, W2, Ws1, Ws3, Ws2):
    raise NotImplementedError("write your pallas kernel here")


# speedup vs baseline: 1.2013x; 1.2013x over previous
import functools

import jax
import jax.numpy as jnp
from jax.experimental import pallas as pl
from jax.experimental.pallas import tpu as pltpu

T = 2048
D = 1024
F = 512
E = 8
N_GROUP = 4
TOP_K = 2
TOPK_GROUP = 2
ROUTED_SCALING_FACTOR = 2.5
EPAD = 128


def _gate_kernel(x_ref, gwt_ref, bias_ref, rw_ref):
    logits = jnp.dot(x_ref[...].astype(jnp.bfloat16),
                     gwt_ref[...].astype(jnp.bfloat16),
                     preferred_element_type=jnp.float32)
    scores = jax.nn.sigmoid(logits)
    biased = scores + bias_ref[...]

    t = logits.shape[0]
    lane_r = jax.lax.broadcasted_iota(jnp.int32, (EPAD, EPAD), 0)
    lane_c = jax.lax.broadcasted_iota(jnp.int32, (EPAD, EPAD), 1)
    lane1 = jax.lax.broadcasted_iota(jnp.int32, (1, EPAD), 1)

    f32 = jnp.float32
    hi = jax.lax.Precision.HIGHEST

    def pairwise_topk_mask(vals, n, k):
        A = ((lane_c // n) == lane_r).astype(f32)
        B = ((lane_c % n) == lane_r).astype(f32)
        X = jnp.dot(vals, A, preferred_element_type=f32, precision=hi)
        Y = jnp.dot(vals, B, preferred_element_type=f32, precision=hi)
        tie = ((lane1 % n) < (lane1 // n)).astype(f32)
        valid = (lane1 < n * n) & ((lane1 % n) != (lane1 // n))
        beats = jnp.where((Y > X) | ((Y == X) & (tie > 0)), 1.0, 0.0)
        beats = jnp.where(valid, beats, 0.0)
        Csum = ((lane_r // n) == lane_c).astype(f32) * \
               (lane_r < n * n).astype(f32)
        rank = jnp.dot(beats, Csum, preferred_element_type=f32, precision=hi)
        return jnp.where((rank < k) & (lane1 < n), 1.0, 0.0)

    P = (((lane_r // 2) == lane_c) & (lane_r < E)).astype(f32)
    gscore = jnp.dot(biased, P, preferred_element_type=f32, precision=hi)
    gsel = pairwise_topk_mask(gscore, N_GROUP, TOPK_GROUP)
    Q = ((lane_r == (lane_c // 2)) & (lane_c < E)).astype(f32)
    em = jnp.dot(gsel, Q, preferred_element_type=f32, precision=hi)
    masked = jnp.where((em > 0) & (lane1 < E), biased, -1e9)
    sel = pairwise_topk_mask(masked, E, TOP_K)

    picked = sel * scores
    Ones8 = ((lane_r < E) & (lane_c < E)).astype(f32)
    wsum = jnp.dot(picked, Ones8, preferred_element_type=f32,
                   precision=jax.lax.Precision.HIGHEST)
    rw = picked * (ROUTED_SCALING_FACTOR / (wsum + 1e-20))
    shared_lane = ((lane1 == E) | (lane1 == E + 1)).astype(f32)
    rw_ref[...] = jnp.where(lane1 < E, rw, 0.0) + shared_lane


def _expert_kernel(x_ref, w1_ref, w3_ref, w2_ref, rw_ref, out_ref):
    e = pl.program_id(0)

    @pl.when(e == 0)
    def _init():
        out_ref[...] = jnp.zeros_like(out_ref)

    lane = jax.lax.broadcasted_iota(jnp.int32, (1, EPAD), 1)
    onehot = (lane == e).astype(jnp.float32)
    scale = jnp.sum(rw_ref[...] * onehot, axis=1, keepdims=True)

    xs = x_ref[...]
    w1 = w1_ref[0]
    w3 = w3_ref[0]
    h1 = jnp.dot(xs, w1, preferred_element_type=jnp.float32)
    h3 = jnp.dot(xs, w3, preferred_element_type=jnp.float32)
    h = jax.nn.silu(h1) * h3 * scale
    out_ref[...] += jnp.dot(h.astype(jnp.bfloat16), w2_ref[0],
                            preferred_element_type=jnp.float32)


@functools.partial(jax.jit, static_argnames=("interpret",))
def _moe(x, gate_weight, bias, W1, W3, W2, Ws1, Ws3, Ws2, interpret=False):
    gwt = jnp.zeros((D, EPAD), jnp.float32).at[:, :E].set(gate_weight.T)
    bias_row = jnp.zeros((1, EPAD), jnp.float32).at[0, :E].set(bias)

    rw = pl.pallas_call(
        _gate_kernel,
        out_shape=jax.ShapeDtypeStruct((T, EPAD), jnp.float32),
        interpret=interpret,
    )(x, gwt, bias_row)

    FS = Ws1.shape[1]
    ns = FS // F
    W1a = jnp.concatenate(
        [W1, Ws1.reshape(D, ns, F).transpose(1, 0, 2)], axis=0)
    W3a = jnp.concatenate(
        [W3, Ws3.reshape(D, ns, F).transpose(1, 0, 2)], axis=0)
    W2a = jnp.concatenate([W2, Ws2.reshape(ns, F, D)], axis=0)
    EA = E + ns

    out = pl.pallas_call(
        _expert_kernel,
        grid=(EA,),
        in_specs=[
            pl.BlockSpec((T, D), lambda e: (0, 0)),
            pl.BlockSpec((1, D, F), lambda e: (e, 0, 0)),
            pl.BlockSpec((1, D, F), lambda e: (e, 0, 0)),
            pl.BlockSpec((1, F, D), lambda e: (e, 0, 0)),
            pl.BlockSpec((T, EPAD), lambda e: (0, 0)),
        ],
        out_specs=pl.BlockSpec((T, D), lambda e: (0, 0)),
        out_shape=jax.ShapeDtypeStruct((T, D), jnp.float32),
        interpret=interpret,
    )(x.astype(jnp.bfloat16), W1a.astype(jnp.bfloat16),
      W3a.astype(jnp.bfloat16), W2a.astype(jnp.bfloat16), rw)
    return out


def kernel(hidden_states, gate_weight, e_score_correction_bias,
           W1, W3, W2, Ws1, Ws3, Ws2):
    return _moe(hidden_states, gate_weight, e_score_correction_bias,
                W1, W3, W2, Ws1, Ws3, Ws2)


# speedup vs baseline: 1.4992x; 1.2480x over previous
import functools

import jax
import jax.numpy as jnp
from jax.experimental import pallas as pl
from jax.experimental.pallas import tpu as pltpu

T = 2048
D = 1024
F = 512
E = 8
N_GROUP = 4
TOP_K = 2
TOPK_GROUP = 2
ROUTED_SCALING_FACTOR = 2.5
EPAD = 128


def _gate_kernel(lg_ref, bias_ref, rw_ref):
    logits = lg_ref[...]
    scores = jax.nn.sigmoid(logits)
    biased = scores + bias_ref[...]

    lane_r = jax.lax.broadcasted_iota(jnp.int32, (EPAD, EPAD), 0)
    lane_c = jax.lax.broadcasted_iota(jnp.int32, (EPAD, EPAD), 1)
    lane1 = jax.lax.broadcasted_iota(jnp.int32, (1, EPAD), 1)

    f32 = jnp.float32
    hi = jax.lax.Precision.HIGHEST

    def pairwise_topk_mask(vals, n, k):
        A = ((lane_c // n) == lane_r).astype(f32)
        B = ((lane_c % n) == lane_r).astype(f32)
        X = jnp.dot(vals, A, preferred_element_type=f32, precision=hi)
        Y = jnp.dot(vals, B, preferred_element_type=f32, precision=hi)
        tie = ((lane1 % n) < (lane1 // n)).astype(f32)
        valid = (lane1 < n * n) & ((lane1 % n) != (lane1 // n))
        beats = jnp.where((Y > X) | ((Y == X) & (tie > 0)), 1.0, 0.0)
        beats = jnp.where(valid, beats, 0.0)
        Csum = ((lane_r // n) == lane_c).astype(f32) * \
               (lane_r < n * n).astype(f32)
        rank = jnp.dot(beats, Csum, preferred_element_type=f32, precision=hi)
        return jnp.where((rank < k) & (lane1 < n), 1.0, 0.0)

    P = (((lane_r // 2) == lane_c) & (lane_r < E)).astype(f32)
    gscore = jnp.dot(biased, P, preferred_element_type=f32, precision=hi)
    gsel = pairwise_topk_mask(gscore, N_GROUP, TOPK_GROUP)
    Q = ((lane_r == (lane_c // 2)) & (lane_c < E)).astype(f32)
    em = jnp.dot(gsel, Q, preferred_element_type=f32, precision=hi)
    masked = jnp.where((em > 0) & (lane1 < E), biased, -1e9)
    sel = pairwise_topk_mask(masked, E, TOP_K)

    picked = sel * scores
    Ones8 = ((lane_r < E) & (lane_c < E)).astype(f32)
    wsum = jnp.dot(picked, Ones8, preferred_element_type=f32, precision=hi)
    rw = picked * (ROUTED_SCALING_FACTOR / (wsum + 1e-20))
    shared_lane = ((lane1 == E) | (lane1 == E + 1)).astype(f32)
    rw_ref[...] = jnp.where(lane1 < E, rw, 0.0) + shared_lane


def _expert_kernel(x_ref, w1_ref, w3_ref, w2_ref,
                   ws1_ref, ws3_ref, ws2_ref, rw_ref, out_ref,
                   w1b, w3b, w2b):
    e = pl.program_id(0)

    @pl.when(e == 0)
    def _init():
        out_ref[...] = jnp.zeros_like(out_ref)

    @pl.when(e < E)
    def _load_routed():
        w1b[...] = w1_ref[0].astype(jnp.bfloat16)
        w3b[...] = w3_ref[0].astype(jnp.bfloat16)
        w2b[...] = w2_ref[0].astype(jnp.bfloat16)

    @pl.when(e >= E)
    def _load_shared():
        w1b[...] = ws1_ref[0].astype(jnp.bfloat16)
        w3b[...] = ws3_ref[0].astype(jnp.bfloat16)
        w2b[...] = ws2_ref[0].astype(jnp.bfloat16)

    lane = jax.lax.broadcasted_iota(jnp.int32, (1, EPAD), 1)
    onehot = (lane == e).astype(jnp.float32)
    scale = jnp.sum(rw_ref[...] * onehot, axis=1, keepdims=True)

    xs = x_ref[...]
    h1 = jnp.dot(xs, w1b[...], preferred_element_type=jnp.float32)
    h3 = jnp.dot(xs, w3b[...], preferred_element_type=jnp.float32)
    h = jax.nn.silu(h1) * h3 * scale
    out_ref[...] += jnp.dot(h.astype(jnp.bfloat16), w2b[...],
                            preferred_element_type=jnp.float32)


@functools.partial(jax.jit, static_argnames=("interpret",))
def _moe(x, gate_weight, bias, W1, W3, W2, Ws1, Ws3, Ws2, interpret=False):
    logits = jnp.matmul(x, gate_weight.T)
    lg = jnp.zeros((T, EPAD), jnp.float32).at[:, :E].set(logits)
    bias_row = jnp.zeros((1, EPAD), jnp.float32).at[0, :E].set(bias)

    rw = pl.pallas_call(
        _gate_kernel,
        out_shape=jax.ShapeDtypeStruct((T, EPAD), jnp.float32),
        interpret=interpret,
    )(lg, bias_row)

    FS = Ws1.shape[1]
    ns = FS // F
    Ws1r = Ws1.reshape(D, ns, F).transpose(1, 0, 2)
    Ws3r = Ws3.reshape(D, ns, F).transpose(1, 0, 2)
    Ws2r = Ws2.reshape(ns, F, D)
    EA = E + ns

    out = pl.pallas_call(
        _expert_kernel,
        grid=(EA,),
        in_specs=[
            pl.BlockSpec((T, D), lambda e: (0, 0)),
            pl.BlockSpec((1, D, F), lambda e: (jnp.minimum(e, E - 1), 0, 0)),
            pl.BlockSpec((1, D, F), lambda e: (jnp.minimum(e, E - 1), 0, 0)),
            pl.BlockSpec((1, F, D), lambda e: (jnp.minimum(e, E - 1), 0, 0)),
            pl.BlockSpec((1, D, F), lambda e: (jnp.maximum(e - E, 0), 0, 0)),
            pl.BlockSpec((1, D, F), lambda e: (jnp.maximum(e - E, 0), 0, 0)),
            pl.BlockSpec((1, F, D), lambda e: (jnp.maximum(e - E, 0), 0, 0)),
            pl.BlockSpec((T, EPAD), lambda e: (0, 0)),
        ],
        out_specs=pl.BlockSpec((T, D), lambda e: (0, 0)),
        out_shape=jax.ShapeDtypeStruct((T, D), jnp.float32),
        scratch_shapes=[
            pltpu.VMEM((D, F), jnp.bfloat16),
            pltpu.VMEM((D, F), jnp.bfloat16),
            pltpu.VMEM((F, D), jnp.bfloat16),
        ],
        interpret=interpret,
    )(x.astype(jnp.bfloat16), W1, W3, W2, Ws1r, Ws3r, Ws2r, rw)
    return out


def kernel(hidden_states, gate_weight, e_score_correction_bias,
           W1, W3, W2, Ws1, Ws3, Ws2):
    return _moe(hidden_states, gate_weight, e_score_correction_bias,
                W1, W3, W2, Ws1, Ws3, Ws2)
